# baseline (speedup 1.0000x reference)
import jax
import jax.numpy as jnp
from jax import lax
from jax.experimental import pallas as pl
from jax.experimental.pallas import tpu as pltpu, tpu_sc as plsc

N_IN = 200000
N_OUT = 50000
D = 256
L = 16
NJ = D // L
S = 200
NCH = N_OUT // S
R = 128
NC = 2
NS = 16
NW = NC * NS
OFF_PAD = 272


def _make_body(half):
  def _sc_body(feat_hbm, ids_hbm, off_hbm, out_hbm, xbuf, accbuf, offs_vm, idbuf, sem):
    wid = lax.axis_index("s")
    pltpu.sync_copy(off_hbm, offs_vm)
    neg = jnp.full((L,), -jnp.inf, dtype=jnp.float32)
    for j in range(NJ):
        xbuf[2 * R, pl.ds(j * L, L)] = neg

    def chunk_body(i, carry):
        c = i * NS + wid + half * (NCH // 2)

        @pl.when(c < (half + 1) * (NCH // 2))
        def _():
            base = pl.multiple_of(c * S, 8)
            offv = offs_vm[pl.ds(c, L)]
            r_lo = offv[0]
            r_hi = offv[1]
            start0 = (r_lo // 8) * 8
            nsteps = (r_hi - start0 + R - 1) // R

            def issue(s):
                st = pl.multiple_of(jnp.minimum(start0 + s * R, N_IN - R), 8)
                b = lax.rem(s, 2)
                pltpu.make_async_copy(
                    feat_hbm.at[pl.ds(st, R), :],
                    xbuf.at[pl.ds(b * R, R), :],
                    sem.at[b],
                ).start()
                pltpu.make_async_copy(
                    ids_hbm.at[pl.ds(st, R)],
                    idbuf.at[pl.ds(b * R, R)],
                    sem.at[b],
                ).start()

            def drain(s):
                st = pl.multiple_of(jnp.minimum(start0 + s * R, N_IN - R), 8)
                b = lax.rem(s, 2)
                pltpu.make_async_copy(
                    feat_hbm.at[pl.ds(st, R), :],
                    xbuf.at[pl.ds(b * R, R), :],
                    sem.at[b],
                ).wait()
                pltpu.make_async_copy(
                    ids_hbm.at[pl.ds(st, R)],
                    idbuf.at[pl.ds(b * R, R)],
                    sem.at[b],
                ).wait()

            issue(0)

            def step_body(s, carry2):
                drain(s)
                b = lax.rem(s, 2)

                @pl.when(s + 1 < nsteps)
                def _():
                    issue(s + 1)

                st = jnp.minimum(start0 + s * R, N_IN - R)
                lo = start0 + s * R - st
                xb = b * R

                REDIR = 2 * R

                idvec0 = idbuf[pl.ds(b * R, L)]
                sid0 = idvec0[0]
                proc0 = (0 >= lo) & (sid0 >= base) & (sid0 < base + S)
                xo0 = jnp.where(proc0, xb, REDIR)
                xs0 = [xbuf[xo0, pl.ds(j * L, L)] for j in range(NJ)]

                def grp_body(g, carry3):
                    cur, sidc, procc = carry3[0], carry3[1], carry3[2]
                    acc = list(carry3[3:3 + NJ])
                    xsc = list(carry3[3 + NJ:])
                    idvec = idbuf[pl.ds(b * R + g * L, L)]
                    idvecn = idbuf[pl.ds(b * R + g * L + L, L)]
                    for jj in range(L):
                        r = g * L + jj
                        sidn = idvec[jj + 1] if jj < L - 1 else idvecn[0]
                        procn = (r + 1 >= lo) & (sidn >= base) & (sidn < base + S)
                        xon = jnp.where(procn, xb + r + 1, REDIR)
                        reinit = procc & (sidc != cur)
                        ao = jnp.clip(cur - base, 0, S) * D
                        xsn = []
                        for j in range(NJ):
                            xsn.append(xbuf[xon, pl.ds(j * L, L)])
                            accbuf[pl.ds(ao + j * L, L)] = acc[j]
                            acc[j] = jnp.where(
                                reinit, xsc[j], jnp.maximum(acc[j], xsc[j])
                            )
                        cur = jnp.where(procc, sidc, cur)
                        sidc, procc, xsc = sidn, procn, xsn
                    return (cur, sidc, procc) + tuple(acc) + tuple(xsc)

                res = lax.fori_loop(
                    0,
                    R // L,
                    grp_body,
                    (carry2[0], sid0, proc0) + tuple(carry2[1:]) + tuple(xs0),
                )
                return (res[0],) + tuple(res[3:3 + NJ])

            init = (base + S,) + tuple(neg for _ in range(NJ))
            fin = lax.fori_loop(0, nsteps, step_body, init)
            cur = fin[0]
            ao = jnp.clip(cur - base, 0, S) * D
            for j in range(NJ):
                accbuf[pl.ds(ao + j * L, L)] = fin[1 + j]
            lbase = base - half * (N_OUT // 2)
            pltpu.sync_copy(
                accbuf.at[pl.ds(0, S * D)],
                out_hbm.at[pl.ds(pl.multiple_of(lbase, 8) * D, S * D)],
            )

        return carry

    lax.fori_loop(0, (NCH // 2 + NS - 1) // NS, chunk_body, 0)
  return _sc_body


@jax.jit
def kernel(features, out_site_ids):
    bounds = jnp.arange(NCH + 1, dtype=jnp.int32) * S
    offs = jnp.searchsorted(out_site_ids, bounds, side="left").astype(jnp.int32)
    offs = jnp.concatenate(
        [offs, jnp.full((OFF_PAD - NCH - 1,), N_IN, dtype=jnp.int32)]
    )
    mesh = plsc.VectorSubcoreMesh(
        core_axis_name="c", subcore_axis_name="s", num_cores=1
    )
    scratch = [
        pltpu.VMEM((2 * R + 1, D), jnp.float32),
        pltpu.VMEM(((S + 1) * D,), jnp.float32),
        pltpu.VMEM((OFF_PAD,), jnp.int32),
        pltpu.VMEM((2 * R + L,), jnp.int32),
        pltpu.SemaphoreType.DMA((2,)),
    ]
    halves = [
        pl.kernel(
            _make_body(h),
            out_type=jax.ShapeDtypeStruct((N_OUT // 2 * D,), jnp.float32),
            mesh=mesh,
            scratch_types=scratch,
        )(features, out_site_ids, offs)
        for h in (0, 1)
    ]
    return jnp.concatenate(halves).reshape(N_OUT, D)


# speedup vs baseline: 1.4766x; 1.4766x over previous
import jax
import jax.numpy as jnp
from jax import lax
from jax.experimental import pallas as pl
from jax.experimental.pallas import tpu as pltpu, tpu_sc as plsc

N_IN = 200000
N_OUT = 50000
D = 256
L = 16
NJ = D // L
S = 200
NCH = N_OUT // S
R = 128
NC = 2
NS = 16
NW = NC * NS
OFF_PAD = 272


def _sc_body(feat_hbm, ids_hbm, off_hbm, out_hbm, xbuf, accbuf, offs_vm, idbuf, sem):
    wid = lax.axis_index("s") * NC + lax.axis_index("c")
    pltpu.sync_copy(off_hbm, offs_vm)
    neg = jnp.full((L,), -jnp.inf, dtype=jnp.float32)
    for j in range(NJ):
        xbuf[2 * R, pl.ds(j * L, L)] = neg

    def chunk_body(i, carry):
        c = i * NW + wid

        @pl.when(c < NCH)
        def _():
            base = pl.multiple_of(c * S, 8)
            offv = offs_vm[pl.ds(c, L)]
            r_lo = offv[0]
            r_hi = offv[1]
            start0 = (r_lo // 8) * 8
            nsteps = (r_hi - start0 + R - 1) // R

            def issue(s):
                st = pl.multiple_of(jnp.minimum(start0 + s * R, N_IN - R), 8)
                b = lax.rem(s, 2)
                pltpu.make_async_copy(
                    feat_hbm.at[pl.ds(st, R), :],
                    xbuf.at[pl.ds(b * R, R), :],
                    sem.at[b],
                ).start()
                pltpu.make_async_copy(
                    ids_hbm.at[pl.ds(st, R)],
                    idbuf.at[pl.ds(b * R, R)],
                    sem.at[b],
                ).start()

            def drain(s):
                st = pl.multiple_of(jnp.minimum(start0 + s * R, N_IN - R), 8)
                b = lax.rem(s, 2)
                pltpu.make_async_copy(
                    feat_hbm.at[pl.ds(st, R), :],
                    xbuf.at[pl.ds(b * R, R), :],
                    sem.at[b],
                ).wait()
                pltpu.make_async_copy(
                    ids_hbm.at[pl.ds(st, R)],
                    idbuf.at[pl.ds(b * R, R)],
                    sem.at[b],
                ).wait()

            issue(0)

            def step_body(s, carry2):
                drain(s)
                b = lax.rem(s, 2)

                @pl.when(s + 1 < nsteps)
                def _():
                    issue(s + 1)

                st = jnp.minimum(start0 + s * R, N_IN - R)
                lo = start0 + s * R - st
                xb = b * R

                REDIR = 2 * R

                idvec0 = idbuf[pl.ds(b * R, L)]
                sid0 = idvec0[0]
                proc0 = (0 >= lo) & (sid0 >= base) & (sid0 < base + S)
                xo0 = jnp.where(proc0, xb, REDIR)
                xs0 = [xbuf[xo0, pl.ds(j * L, L)] for j in range(NJ)]

                def grp_body(g, carry3):
                    cur, sidc, procc = carry3[0], carry3[1], carry3[2]
                    acc = list(carry3[3:3 + NJ])
                    xsc = list(carry3[3 + NJ:])
                    idvec = idbuf[pl.ds(b * R + g * L, L)]
                    idvecn = idbuf[pl.ds(b * R + g * L + L, L)]
                    for jj in range(L):
                        r = g * L + jj
                        sidn = idvec[jj + 1] if jj < L - 1 else idvecn[0]
                        procn = (r + 1 >= lo) & (sidn >= base) & (sidn < base + S)
                        xon = jnp.where(procn, xb + r + 1, REDIR)
                        reinit = procc & (sidc != cur)
                        ao = jnp.clip(cur - base, 0, S) * D
                        xsn = []
                        for j in range(NJ):
                            xsn.append(xbuf[xon, pl.ds(j * L, L)])
                            accbuf[pl.ds(ao + j * L, L)] = acc[j]
                            acc[j] = jnp.where(
                                reinit, xsc[j], jnp.maximum(acc[j], xsc[j])
                            )
                        cur = jnp.where(procc, sidc, cur)
                        sidc, procc, xsc = sidn, procn, xsn
                    return (cur, sidc, procc) + tuple(acc) + tuple(xsc)

                res = lax.fori_loop(
                    0,
                    R // L,
                    grp_body,
                    (carry2[0], sid0, proc0) + tuple(carry2[1:]) + tuple(xs0),
                )
                return (res[0],) + tuple(res[3:3 + NJ])

            init = (base + S,) + tuple(neg for _ in range(NJ))
            fin = lax.fori_loop(0, nsteps, step_body, init)
            cur = fin[0]
            ao = jnp.clip(cur - base, 0, S) * D
            for j in range(NJ):
                accbuf[pl.ds(ao + j * L, L)] = fin[1 + j]
            pltpu.sync_copy(
                accbuf.at[pl.ds(0, S * D)], out_hbm.at[pl.ds(base * D, S * D)]
            )

        return carry

    lax.fori_loop(0, (NCH + NW - 1) // NW, chunk_body, 0)


@jax.jit
def kernel(features, out_site_ids):
    bounds = jnp.arange(NCH + 1, dtype=jnp.int32) * S
    offs = jnp.searchsorted(out_site_ids, bounds, side="left").astype(jnp.int32)
    offs = jnp.concatenate(
        [offs, jnp.full((OFF_PAD - NCH - 1,), N_IN, dtype=jnp.int32)]
    )
    mesh = plsc.VectorSubcoreMesh(core_axis_name="c", subcore_axis_name="s")
    out = pl.kernel(
        _sc_body,
        out_type=jax.ShapeDtypeStruct((N_OUT * D,), jnp.float32),
        mesh=mesh,
        scratch_types=[
            pltpu.VMEM((2 * R + 1, D), jnp.float32),
            pltpu.VMEM(((S + 1) * D,), jnp.float32),
            pltpu.VMEM((OFF_PAD,), jnp.int32),
            pltpu.VMEM((2 * R + L,), jnp.int32),
            pltpu.SemaphoreType.DMA((2,)),
        ],
    )(features, out_site_ids, offs)
    return out.reshape(N_OUT, D)


# speedup vs baseline: 1.5385x; 1.0419x over previous
import jax
import jax.numpy as jnp
from jax import lax
from jax.experimental import pallas as pl
from jax.experimental.pallas import tpu as pltpu, tpu_sc as plsc

N_IN = 200000
N_OUT = 50000
D = 256
L = 16
NJ = D // L
S = 200
NCH = N_OUT // S
R = 128
NC = 2
NS = 16
NW = NC * NS
OFF_PAD = 456


def _sc_body(feat_hbm, ids_hbm, off_hbm, out_hbm, xbuf, accbuf, offs_vm, idbuf, sem):
    wid = lax.axis_index("s") * NC + lax.axis_index("c")
    pltpu.sync_copy(off_hbm, offs_vm)
    neg = jnp.full((L,), -jnp.inf, dtype=jnp.float32)
    for j in range(NJ):
        xbuf[2 * R, pl.ds(j * L, L)] = neg

    def chunk_start0(c):
        offv = offs_vm[pl.ds(c, L)]
        return (offv[0] // 8) * 8, offv[1]

    def issue_at(start0, s, b):
        st = pl.multiple_of(jnp.minimum(start0 + s * R, N_IN - R), 8)
        pltpu.make_async_copy(
            feat_hbm.at[pl.ds(st, R), :],
            xbuf.at[pl.ds(b * R, R), :],
            sem.at[b],
        ).start()
        pltpu.make_async_copy(
            ids_hbm.at[pl.ds(st, R)],
            idbuf.at[pl.ds(b * R, R)],
            sem.at[b],
        ).start()

    def drain_at(start0, s, b):
        st = pl.multiple_of(jnp.minimum(start0 + s * R, N_IN - R), 8)
        pltpu.make_async_copy(
            feat_hbm.at[pl.ds(st, R), :],
            xbuf.at[pl.ds(b * R, R), :],
            sem.at[b],
        ).wait()
        pltpu.make_async_copy(
            ids_hbm.at[pl.ds(st, R)],
            idbuf.at[pl.ds(b * R, R)],
            sem.at[b],
        ).wait()

    start0_w, _ = chunk_start0(wid)
    issue_at(start0_w, 0, 0)

    def chunk_body(i, gp_in):
        c = i * NW + wid
        gp_box = [gp_in]

        @pl.when(c < NCH)
        def _():
            gp = gp_in
            base = pl.multiple_of(c * S, 8)
            start0, r_hi = chunk_start0(c)
            nsteps = (r_hi - start0 + R - 1) // R

            def step_body(s, carry2):
                b = lax.rem(gp + s, 2)
                drain_at(start0, s, b)
                bn = lax.rem(gp + s + 1, 2)

                @pl.when(s + 1 < nsteps)
                def _():
                    issue_at(start0, s + 1, bn)

                @pl.when((s + 1 == nsteps) & (c + NW < NCH))
                def _():
                    nstart0, _ = chunk_start0(c + NW)
                    issue_at(nstart0, 0, bn)

                st = jnp.minimum(start0 + s * R, N_IN - R)
                lo = start0 + s * R - st
                xb = b * R

                REDIR = 2 * R

                idvec0 = idbuf[pl.ds(b * R, L)]
                sid0 = idvec0[0]
                proc0 = (0 >= lo) & (sid0 >= base) & (sid0 < base + S)
                xo0 = jnp.where(proc0, xb, REDIR)
                xs0 = [xbuf[xo0, pl.ds(j * L, L)] for j in range(NJ)]

                def grp_body(g, carry3):
                    cur, sidc, procc = carry3[0], carry3[1], carry3[2]
                    acc = list(carry3[3:3 + NJ])
                    xsc = list(carry3[3 + NJ:])
                    idvec = idbuf[pl.ds(b * R + g * L, L)]
                    idvecn = idbuf[pl.ds(b * R + g * L + L, L)]
                    for jj in range(L):
                        r = g * L + jj
                        sidn = idvec[jj + 1] if jj < L - 1 else idvecn[0]
                        procn = (r + 1 >= lo) & (sidn >= base) & (sidn < base + S)
                        xon = jnp.where(procn, xb + r + 1, REDIR)
                        reinit = procc & (sidc != cur)
                        ao = jnp.clip(cur - base, 0, S) * D
                        xsn = []
                        for j in range(NJ):
                            xsn.append(xbuf[xon, pl.ds(j * L, L)])
                            accbuf[pl.ds(ao + j * L, L)] = acc[j]
                            acc[j] = jnp.where(
                                reinit, xsc[j], jnp.maximum(acc[j], xsc[j])
                            )
                        cur = jnp.where(procc, sidc, cur)
                        sidc, procc, xsc = sidn, procn, xsn
                    return (cur, sidc, procc) + tuple(acc) + tuple(xsc)

                res = lax.fori_loop(
                    0,
                    R // L,
                    grp_body,
                    (carry2[0], sid0, proc0) + tuple(carry2[1:]) + tuple(xs0),
                )
                return (res[0],) + tuple(res[3:3 + NJ])

            init = (base + S,) + tuple(neg for _ in range(NJ))
            fin = lax.fori_loop(0, nsteps, step_body, init)
            cur = fin[0]
            ao = jnp.clip(cur - base, 0, S) * D
            for j in range(NJ):
                accbuf[pl.ds(ao + j * L, L)] = fin[1 + j]
            pltpu.sync_copy(
                accbuf.at[pl.ds(0, S * D)], out_hbm.at[pl.ds(base * D, S * D)]
            )

        start0_u, r_hi_u = chunk_start0(jnp.minimum(c, NCH - 1))
        nsteps_u = (r_hi_u - start0_u + R - 1) // R
        return jnp.where(c < NCH, gp_in + nsteps_u, gp_in)

    lax.fori_loop(0, (NCH + NW - 1) // NW, chunk_body, 0)


@jax.jit
def kernel(features, out_site_ids):
    bounds = jnp.arange(NCH + 1, dtype=jnp.int32) * S
    offs = jnp.searchsorted(out_site_ids, bounds, side="left").astype(jnp.int32)
    offs = jnp.concatenate(
        [offs, jnp.full((OFF_PAD - NCH - 1,), N_IN, dtype=jnp.int32)]
    )
    mesh = plsc.VectorSubcoreMesh(core_axis_name="c", subcore_axis_name="s")
    out = pl.kernel(
        _sc_body,
        out_type=jax.ShapeDtypeStruct((N_OUT * D,), jnp.float32),
        mesh=mesh,
        scratch_types=[
            pltpu.VMEM((2 * R + 1, D), jnp.float32),
            pltpu.VMEM(((S + 1) * D,), jnp.float32),
            pltpu.VMEM((OFF_PAD,), jnp.int32),
            pltpu.VMEM((2 * R + L,), jnp.int32),
            pltpu.SemaphoreType.DMA((2,)),
        ],
    )(features, out_site_ids, offs)
    return out.reshape(N_OUT, D)
